# baseline (speedup 1.0000x reference)
import functools

import jax
import jax.numpy as jnp
from jax.experimental import pallas as pl

_NC = 68
_BF = jnp.bfloat16


_CS = 2048


def _conv_kern(x_ref, w_ref, o_ref, *, taps, n, act, odt):
    for c0 in range(0, n, _CS):
        m = min(_CS, n - c0)
        acc = None
        for k, off in enumerate(taps):
            xk = x_ref[0, c0 + off:c0 + off + m, :]
            p = jnp.dot(xk, w_ref[k], preferred_element_type=jnp.float32)
            acc = p if acc is None else acc + p
        if act == "relu":
            acc = jnp.maximum(acc, 0.0)
        elif act == "tanh":
            acc = jnp.tanh(acc)
        o_ref[0, c0:c0 + m, :] = acc.astype(odt)


def _conv_skip_kern(x_ref, w_ref, s_ref, o_ref, *, taps, n, odt):
    for c0 in range(0, n, _CS):
        m = min(_CS, n - c0)
        acc = None
        for k, off in enumerate(taps):
            xk = x_ref[0, c0 + off:c0 + off + m, :]
            p = jnp.dot(xk, w_ref[k], preferred_element_type=jnp.float32)
            acc = p if acc is None else acc + p
        acc = jnp.maximum(acc, 0.0) + s_ref[0, c0:c0 + m, :].astype(jnp.float32)
        o_ref[0, c0:c0 + m, :] = acc.astype(odt)


def _conv_s2_kern(x00, x01, x10, x11, w_ref, o_ref, *, wq, n, odt):
    phases = (x00, x01, x10, x11)
    for c0 in range(0, n, _CS):
        m = min(_CS, n - c0)
        acc = None
        for dy in range(3):
            for dx in range(3):
                ref = phases[(dy & 1) * 2 + (dx & 1)]
                off = c0 + (wq if dy == 2 else 0) + (1 if dx == 2 else 0)
                xk = ref[0, off:off + m, :]
                p = jnp.dot(xk, w_ref[dy * 3 + dx],
                            preferred_element_type=jnp.float32)
                acc = p if acc is None else acc + p
        o_ref[0, c0:c0 + m, :] = jnp.maximum(acc, 0.0).astype(odt)


def _group(x, g, e):
    b, sp, c = x.shape
    x = x.reshape(b // g, g * sp, c)
    return jnp.pad(x, ((0, 0), (0, e), (0, 0))) if e else x


def _w9(w):
    co, ci = w.shape[0], w.shape[1]
    return w.transpose(2, 3, 1, 0).reshape(9, ci, co).astype(_BF)


def _up(t, f):
    b, h, w, c = t.shape
    t = jnp.broadcast_to(t[:, :, None, :, None, :], (b, h, f, w, f, c))
    return t.reshape(b, h * f, w * f, c)


def _conv_s1(x, w9, g, act="relu", skip=None, odt=_BF):
    b, h, w, c = x.shape
    wp = w + 2
    sp = (h + 2) * wp + 2
    e = 2 * wp + 2
    n = g * sp
    nb = b // g
    co = w9.shape[2]
    xf = jnp.pad(x, ((0, 0), (1, 1), (1, 1), (0, 0))).reshape(b, (h + 2) * wp, c)
    xf = jnp.pad(xf, ((0, 0), (0, 2), (0, 0)))
    xg = _group(xf, g, e)
    taps = tuple(dy * wp + dx for dy in range(3) for dx in range(3))
    in_specs = [
        pl.BlockSpec((1, n + e, c), lambda i: (i, 0, 0)),
        pl.BlockSpec(w9.shape, lambda i: (0, 0, 0)),
    ]
    args = [xg, w9]
    if skip is None:
        kern = functools.partial(_conv_kern, taps=taps, n=n, act=act, odt=odt)
    else:
        kern = functools.partial(_conv_skip_kern, taps=taps, n=n, odt=odt)
        sf = jnp.pad(skip, ((0, 0), (0, 0), (0, 2), (0, 0))).reshape(b, h * wp, co)
        sf = jnp.pad(sf, ((0, 0), (0, sp - h * wp), (0, 0))).astype(_BF)
        in_specs.append(pl.BlockSpec((1, n, co), lambda i: (i, 0, 0)))
        args.append(_group(sf, g, 0))
    out = pl.pallas_call(
        kern,
        grid=(nb,),
        in_specs=in_specs,
        out_specs=pl.BlockSpec((1, n, co), lambda i: (i, 0, 0)),
        out_shape=jax.ShapeDtypeStruct((nb, n, co), odt),
    )(*args)
    o = out.reshape(b, sp, co)[:, :h * wp, :].reshape(b, h, wp, co)
    return o[:, :, :w, :]


def _conv_s2(x, w9, g, odt=_BF):
    b, h, w, c = x.shape
    ho, wo = h // 2, w // 2
    wq = wo + 1
    sp = (ho + 1) * wq
    e = wq + 1
    n = g * sp
    nb = b // g
    co = w9.shape[2]
    xp = jnp.pad(x, ((0, 0), (0, 2), (0, 2), (0, 0)))
    phases = []
    for p in range(2):
        for q in range(2):
            t = xp[:, p:p + 2 * (ho + 1):2, q:q + 2 * wq:2, :].reshape(b, sp, c)
            phases.append(_group(t, g, e))
    kern = functools.partial(_conv_s2_kern, wq=wq, n=n, odt=odt)
    pspec = pl.BlockSpec((1, n + e, c), lambda i: (i, 0, 0))
    out = pl.pallas_call(
        kern,
        grid=(nb,),
        in_specs=[pspec, pspec, pspec, pspec,
                  pl.BlockSpec(w9.shape, lambda i: (0, 0, 0))],
        out_specs=pl.BlockSpec((1, n, co), lambda i: (i, 0, 0)),
        out_shape=jax.ShapeDtypeStruct((nb, n, co), odt),
    )(*phases, w9)
    o = out.reshape(b, sp, co)[:, :ho * wq, :].reshape(b, ho, wq, co)
    return o[:, :, :wo, :]


def _read_kern(rk_ref, tk_ref, x_ref, o_ref):
    rk = rk_ref[0:1, :]
    m = None
    for d in range(3):
        md = (tk_ref[:, d:d + 1] == rk).astype(jnp.float32)
        cnt = jnp.sum(md, axis=1, keepdims=True)
        term = md / jnp.maximum(cnt, 1.0)
        m = term if m is None else m + term
    m = m * (1.0 / 3.0)
    nb = x_ref.shape[1]
    for c0 in range(0, nb, _CS):
        w = min(_CS, nb - c0)
        o_ref[:, c0:c0 + w] = jnp.dot(
            m, x_ref[:, c0:c0 + w].astype(jnp.float32),
            preferred_element_type=jnp.float32)


def _run_read(rk, tk, x, nb):
    b, ncols = x.shape
    t = tk.shape[0]
    return pl.pallas_call(
        _read_kern,
        grid=(ncols // nb,),
        in_specs=[
            pl.BlockSpec((1, b), lambda j: (0, 0)),
            pl.BlockSpec((t, 3), lambda j: (0, 0)),
            pl.BlockSpec((b, nb), lambda j: (0, j)),
        ],
        out_specs=pl.BlockSpec((t, nb), lambda j: (0, j)),
        out_shape=jax.ShapeDtypeStruct((t, ncols), jnp.float32),
    )(rk, tk, x)


def kernel(ref_fids, ref_decs, ref_imgs, trg_fids, trg_decs,
           We1, We2, We3, We4, Wd1, Wd2, Wd3, Wd4):
    B = ref_imgs.shape[0]
    T = trg_fids.shape[0]

    img = ref_imgs[:, 0].astype(_BF)
    xp = jnp.pad(img, ((0, 0), (0, 1), (0, 1)))
    t9 = jnp.stack(
        [xp[:, dy:dy + 64:2, dx:dx + 64:2].reshape(B, 32 * 32)
         for dy in range(3) for dx in range(3)], axis=-1)
    w1 = We1.transpose(2, 3, 1, 0).reshape(1, 9, 64).astype(_BF)
    g1 = 16
    h = pl.pallas_call(
        functools.partial(_conv_kern, taps=(0,), n=g1 * 1024, act="relu",
                          odt=_BF),
        grid=(B // g1,),
        in_specs=[pl.BlockSpec((1, g1 * 1024, 9), lambda i: (i, 0, 0)),
                  pl.BlockSpec((1, 9, 64), lambda i: (0, 0, 0))],
        out_specs=pl.BlockSpec((1, g1 * 1024, 64), lambda i: (i, 0, 0)),
        out_shape=jax.ShapeDtypeStruct((B // g1, g1 * 1024, 64), _BF),
    )(_group(t9, g1, 0), w1)
    h = h.reshape(B, 32, 32, 64)

    xp_ = jnp.pad(h, ((0, 0), (0, 2), (0, 2), (0, 0)))
    phs = []
    for p_ in range(2):
        for q_ in range(2):
            t_ = xp_[:, p_:p_ + 2 * 17:2, q_:q_ + 2 * 17:2, :].reshape(B, 289, 64)
            phs.append(_group(t_, 16, 18))
    return tuple(phs)

    h = _conv_s2(h, _w9(We2), 16)
    skip = h
    h = _conv_s2(h, _w9(We3), 32)
    last = _conv_s2(h, _w9(We4), 32)

    return last, skip
    rk = (ref_fids.astype(jnp.int32) * _NC
          + ref_decs.astype(jnp.int32)).reshape(1, B)
    tk = (trg_fids.astype(jnp.int32)[:, None] * _NC
          + trg_decs.astype(jnp.int32))
    last_r = _run_read(rk, tk, last.reshape(B, 4 * 4 * 256), 4096)
    skip_r = _run_read(rk, tk, skip.reshape(B, 16 * 16 * 128), 8192)
    last_r = last_r.reshape(T, 4, 4, 256)
    skip_r = skip_r.reshape(T, 16, 16, 128)

    h = _up(last_r, 4).astype(_BF)
    h = _conv_s1(h, _w9(Wd1), 16, skip=skip_r)
    h = _up(h, 2)
    h = _conv_s1(h, _w9(Wd2), 16)
    h = _up(h, 2)
    h = _conv_s1(h, _w9(Wd3), 8)
    h = _conv_s1(h, _w9(Wd4), 2, act="tanh", odt=jnp.float32)
    return h.transpose(0, 3, 1, 2)


# speedup vs baseline: 1.4170x; 1.4170x over previous
import functools

import jax
import jax.numpy as jnp
from jax.experimental import pallas as pl

_NC = 68
_BF = jnp.bfloat16


_CS = 2048


def _conv_kern(x_ref, w_ref, o_ref, *, taps, n, act, odt):
    for c0 in range(0, n, _CS):
        m = min(_CS, n - c0)
        acc = None
        for k, off in enumerate(taps):
            xk = x_ref[0, c0 + off:c0 + off + m, :]
            p = jnp.dot(xk, w_ref[k], preferred_element_type=jnp.float32)
            acc = p if acc is None else acc + p
        if act == "relu":
            acc = jnp.maximum(acc, 0.0)
        elif act == "tanh":
            acc = jnp.tanh(acc)
        o_ref[0, c0:c0 + m, :] = acc.astype(odt)


def _conv_skip_kern(x_ref, w_ref, s_ref, o_ref, *, taps, n, odt):
    for c0 in range(0, n, _CS):
        m = min(_CS, n - c0)
        acc = None
        for k, off in enumerate(taps):
            xk = x_ref[0, c0 + off:c0 + off + m, :]
            p = jnp.dot(xk, w_ref[k], preferred_element_type=jnp.float32)
            acc = p if acc is None else acc + p
        acc = jnp.maximum(acc, 0.0) + s_ref[0, c0:c0 + m, :].astype(jnp.float32)
        o_ref[0, c0:c0 + m, :] = acc.astype(odt)


def _conv_s2_kern(pe_ref, po_ref, wf_ref, wh_ref, o_ref, *, wq, n, ci, odt):
    for c0 in range(0, n, _CS):
        m = min(_CS, n - c0)
        taps = (
            (pe_ref, c0, 0, wf_ref), (pe_ref, c0 + 1, 1, wh_ref),
            (po_ref, c0, 0, wf_ref), (po_ref, c0 + 1, 1, wh_ref),
            (pe_ref, c0 + wq, 0, wf_ref), (pe_ref, c0 + wq + 1, 1, wh_ref),
        )
        acc = None
        for k, (ref, off, half, wref) in enumerate(taps):
            xk = ref[0, off:off + m, :ci] if half else ref[0, off:off + m, :]
            p = jnp.dot(xk, wref[k // 2], preferred_element_type=jnp.float32)
            acc = p if acc is None else acc + p
        o_ref[0, c0:c0 + m, :] = jnp.maximum(acc, 0.0).astype(odt)


def _group(x, g, e):
    b, sp, c = x.shape
    x = x.reshape(b // g, g * sp, c)
    return jnp.pad(x, ((0, 0), (0, e), (0, 0))) if e else x


def _w9(w):
    co, ci = w.shape[0], w.shape[1]
    return w.transpose(2, 3, 1, 0).reshape(9, ci, co).astype(_BF)


def _up(t, f):
    b, h, w, c = t.shape
    t = jnp.broadcast_to(t[:, :, None, :, None, :], (b, h, f, w, f, c))
    return t.reshape(b, h * f, w * f, c)


def _conv_s1(x, w9, g, act="relu", skip=None, odt=_BF):
    b, h, w, c = x.shape
    wp = w + 2
    sp = (h + 2) * wp + 2
    e = 2 * wp + 2
    n = g * sp
    nb = b // g
    co = w9.shape[2]
    xf = jnp.pad(x, ((0, 0), (1, 1), (1, 1), (0, 0))).reshape(b, (h + 2) * wp, c)
    xf = jnp.pad(xf, ((0, 0), (0, 2), (0, 0)))
    xg = _group(xf, g, e)
    taps = tuple(dy * wp + dx for dy in range(3) for dx in range(3))
    in_specs = [
        pl.BlockSpec((1, n + e, c), lambda i: (i, 0, 0)),
        pl.BlockSpec(w9.shape, lambda i: (0, 0, 0)),
    ]
    args = [xg, w9]
    if skip is None:
        kern = functools.partial(_conv_kern, taps=taps, n=n, act=act, odt=odt)
    else:
        kern = functools.partial(_conv_skip_kern, taps=taps, n=n, odt=odt)
        sf = jnp.pad(skip, ((0, 0), (0, 0), (0, 2), (0, 0))).reshape(b, h * wp, co)
        sf = jnp.pad(sf, ((0, 0), (0, sp - h * wp), (0, 0))).astype(_BF)
        in_specs.append(pl.BlockSpec((1, n, co), lambda i: (i, 0, 0)))
        args.append(_group(sf, g, 0))
    out = pl.pallas_call(
        kern,
        grid=(nb,),
        in_specs=in_specs,
        out_specs=pl.BlockSpec((1, n, co), lambda i: (i, 0, 0)),
        out_shape=jax.ShapeDtypeStruct((nb, n, co), odt),
    )(*args)
    o = out.reshape(b, sp, co)[:, :h * wp, :].reshape(b, h, wp, co)
    return o[:, :, :w, :]


def _conv_s2(x, w9, g, odt=_BF):
    b, h, w, c = x.shape
    ho, wo = h // 2, w // 2
    wq = (w + 2) // 2
    hp = (h + 2) // 2
    sp = hp * wq
    e = wq + 1
    n = g * sp
    nb = b // g
    co = w9.shape[2]
    c2 = 2 * c
    xp = jnp.pad(x, ((0, 0), (0, 2), (0, 2), (0, 0)))
    xm = xp.reshape(b, h + 2, wq, c2)
    pe = xm[:, 0::2].reshape(b, sp, c2)
    po = xm[:, 1::2].reshape(b, sp, c2)
    pe = _group(pe, g, e)
    po = _group(po, g, e)
    wf = jnp.concatenate([w9[0::3], w9[1::3]], axis=1)
    wh = w9[2::3]
    kern = functools.partial(_conv_s2_kern, wq=wq, n=n, ci=c, odt=odt)
    pspec = pl.BlockSpec((1, n + e, c2), lambda i: (i, 0, 0))
    out = pl.pallas_call(
        kern,
        grid=(nb,),
        in_specs=[pspec, pspec,
                  pl.BlockSpec(wf.shape, lambda i: (0, 0, 0)),
                  pl.BlockSpec(wh.shape, lambda i: (0, 0, 0))],
        out_specs=pl.BlockSpec((1, n, co), lambda i: (i, 0, 0)),
        out_shape=jax.ShapeDtypeStruct((nb, n, co), odt),
    )(pe, po, wf, wh)
    o = out.reshape(b, sp, co)[:, :ho * wq, :].reshape(b, ho, wq, co)
    return o[:, :, :wo, :]


def _read_kern(rk_ref, tk_ref, x_ref, o_ref):
    rk = rk_ref[0:1, :]
    m = None
    for d in range(3):
        md = (tk_ref[:, d:d + 1] == rk).astype(jnp.float32)
        cnt = jnp.sum(md, axis=1, keepdims=True)
        term = md / jnp.maximum(cnt, 1.0)
        m = term if m is None else m + term
    m = m * (1.0 / 3.0)
    nb = x_ref.shape[1]
    for c0 in range(0, nb, _CS):
        w = min(_CS, nb - c0)
        o_ref[:, c0:c0 + w] = jnp.dot(
            m, x_ref[:, c0:c0 + w].astype(jnp.float32),
            preferred_element_type=jnp.float32)


def _run_read(rk, tk, x, nb):
    b, ncols = x.shape
    t = tk.shape[0]
    return pl.pallas_call(
        _read_kern,
        grid=(ncols // nb,),
        in_specs=[
            pl.BlockSpec((1, b), lambda j: (0, 0)),
            pl.BlockSpec((t, 3), lambda j: (0, 0)),
            pl.BlockSpec((b, nb), lambda j: (0, j)),
        ],
        out_specs=pl.BlockSpec((t, nb), lambda j: (0, j)),
        out_shape=jax.ShapeDtypeStruct((t, ncols), jnp.float32),
    )(rk, tk, x)


def kernel(ref_fids, ref_decs, ref_imgs, trg_fids, trg_decs,
           We1, We2, We3, We4, Wd1, Wd2, Wd3, Wd4):
    B = ref_imgs.shape[0]
    T = trg_fids.shape[0]

    img = ref_imgs[:, 0].astype(_BF)
    xp = jnp.pad(img, ((0, 0), (0, 1), (0, 1)))
    t9 = jnp.stack(
        [xp[:, dy:dy + 64:2, dx:dx + 64:2].reshape(B, 32 * 32)
         for dy in range(3) for dx in range(3)], axis=-1)
    w1 = We1.transpose(2, 3, 1, 0).reshape(1, 9, 64).astype(_BF)
    g1 = 16
    h = pl.pallas_call(
        functools.partial(_conv_kern, taps=(0,), n=g1 * 1024, act="relu",
                          odt=_BF),
        grid=(B // g1,),
        in_specs=[pl.BlockSpec((1, g1 * 1024, 9), lambda i: (i, 0, 0)),
                  pl.BlockSpec((1, 9, 64), lambda i: (0, 0, 0))],
        out_specs=pl.BlockSpec((1, g1 * 1024, 64), lambda i: (i, 0, 0)),
        out_shape=jax.ShapeDtypeStruct((B // g1, g1 * 1024, 64), _BF),
    )(_group(t9, g1, 0), w1)
    h = h.reshape(B, 32, 32, 64)

    h = _conv_s2(h, _w9(We2), 16)
    skip = h
    h = _conv_s2(h, _w9(We3), 32)
    last = _conv_s2(h, _w9(We4), 32)

    rk = (ref_fids.astype(jnp.int32) * _NC
          + ref_decs.astype(jnp.int32)).reshape(1, B)
    tk = (trg_fids.astype(jnp.int32)[:, None] * _NC
          + trg_decs.astype(jnp.int32))
    last_r = _run_read(rk, tk, last.reshape(B, 4 * 4 * 256), 4096)
    skip_r = _run_read(rk, tk, skip.reshape(B, 16 * 16 * 128), 8192)
    last_r = last_r.reshape(T, 4, 4, 256)
    skip_r = skip_r.reshape(T, 16, 16, 128)

    h = _up(last_r, 4).astype(_BF)
    h = _conv_s1(h, _w9(Wd1), 16, skip=skip_r)
    h = _up(h, 2)
    h = _conv_s1(h, _w9(Wd2), 16)
    h = _up(h, 2)
    h = _conv_s1(h, _w9(Wd3), 8)
    h = _conv_s1(h, _w9(Wd4), 2, act="tanh", odt=jnp.float32)
    return h.transpose(0, 3, 1, 2)
